# baseline (speedup 1.0000x reference)
import jax
import jax.numpy as jnp
from jax.experimental import pallas as pl
from jax.experimental.pallas import tpu as pltpu

_UNROLL = 8


def _round_up(v, m):
    return ((v + m - 1) // m) * m


def _tree_sum(vals):
    vals = list(vals)
    while len(vals) > 1:
        nxt = [vals[i] + vals[i + 1] for i in range(0, len(vals) - 1, 2)]
        if len(vals) % 2:
            nxt.append(vals[-1])
        vals = nxt
    return vals[0]


def _make_kernel(block_b, W, H, unroll):
    C = W - 1
    inv_c = 1.0 / C

    def body(ids_ref, in_hbm, out_hbm, o_ref, in_vmem, out_vmem, buf_ref,
             sem_in, sem_out):
        blk = pl.program_id(0)
        base = blk * block_b * W

        cp_out = pltpu.make_async_copy(out_hbm, out_vmem, sem_out)
        cp_in = pltpu.make_async_copy(in_hbm, in_vmem, sem_in)
        cp_out.start()
        cp_in.start()
        cp_out.wait()

        @pl.loop(0, block_b // unroll)
        def _ctx_chunk(ci):
            off0 = base + ci * (unroll * W)
            rows = []
            for u in range(unroll):
                off = off0 + u * W
                ctx = [out_vmem[ids_ref[off + 1 + k], 0] for k in range(C)]
                rows.append(_tree_sum(ctx)[None, :])
            buf_ref[pl.ds(ci * unroll, unroll), :] = jnp.concatenate(rows, 0)

        cp_in.wait()

        @pl.loop(0, block_b // unroll)
        def _tgt_chunk(ci):
            off0 = base + ci * (unroll * W)
            rows = []
            for u in range(unroll):
                off = off0 + u * W
                rows.append(in_vmem[ids_ref[off], 0][None, :])
            t8 = jnp.concatenate(rows, 0)
            r = ci * unroll
            buf_ref[pl.ds(r, unroll), :] = buf_ref[pl.ds(r, unroll), :] * t8

        o_ref[...] = (jnp.sum(buf_ref[...], axis=-1) * inv_c)[None, :]

    return body


def _choose_block(B):
    if B >= 2048 and B % 2048 == 0:
        return B // 2
    if B >= 1024:
        return 512
    return max(_UNROLL, _round_up(B, _UNROLL))


def kernel(x, in_emb, out_emb):
    B, W = x.shape
    C = W - 1
    if C < 1:
        raise ValueError("Skipgram needs at least one context word (W >= 2).")
    V, H = in_emb.shape

    block_b = _choose_block(B)
    grid_b = -(-B // block_b)
    B_pad = grid_b * block_b

    x = x.astype(jnp.int32)
    if B_pad != B:
        x = jnp.pad(x, ((0, B_pad - B), (0, 0)))

    in3 = in_emb.reshape(V, 1, H)
    out3 = out_emb.reshape(V, 1, H)

    table_bytes = 2 * V * H * jnp.dtype(in_emb.dtype).itemsize
    vmem_need = table_bytes + block_b * H * 4 + block_b * 4

    out = pl.pallas_call(
        _make_kernel(block_b, W, H, _UNROLL),
        out_shape=jax.ShapeDtypeStruct((1, B_pad), jnp.float32),
        grid_spec=pltpu.PrefetchScalarGridSpec(
            num_scalar_prefetch=1,
            grid=(grid_b,),
            in_specs=[
                pl.BlockSpec(memory_space=pl.ANY),
                pl.BlockSpec(memory_space=pl.ANY),
            ],
            out_specs=pl.BlockSpec((1, block_b), lambda i, ids: (0, i)),
            scratch_shapes=[
                pltpu.VMEM((V, 1, H), jnp.float32),
                pltpu.VMEM((V, 1, H), jnp.float32),
                pltpu.VMEM((block_b, H), jnp.float32),
                pltpu.SemaphoreType.DMA,
                pltpu.SemaphoreType.DMA,
            ],
        ),
        compiler_params=pltpu.CompilerParams(
            dimension_semantics=("parallel",),
            vmem_limit_bytes=int(min(vmem_need + (16 << 20), 56 << 20)),
        ),
    )(x.reshape(-1), in3, out3)
    return out.reshape(B_pad)[:B]


# speedup vs baseline: 1.2807x; 1.2807x over previous
import jax
import jax.numpy as jnp
from jax.experimental import pallas as pl
from jax.experimental.pallas import tpu as pltpu

_UNROLL = 16


def _round_up(v, m):
    return ((v + m - 1) // m) * m


def _tree_sum(vals):
    vals = list(vals)
    while len(vals) > 1:
        nxt = [vals[i] + vals[i + 1] for i in range(0, len(vals) - 1, 2)]
        if len(vals) % 2:
            nxt.append(vals[-1])
        vals = nxt
    return vals[0]


def _make_kernel(block_b, W, H, unroll):
    C = W - 1
    inv_c = 1.0 / C

    def body(ids_ref, in_hbm, out_hbm, o_ref, in_vmem, out_vmem, buf_ref,
             sem_in, sem_out):
        blk = pl.program_id(0)
        base = blk * block_b * W

        cp_out = pltpu.make_async_copy(out_hbm, out_vmem, sem_out)
        cp_in = pltpu.make_async_copy(in_hbm, in_vmem, sem_in)
        cp_out.start()
        cp_in.start()
        cp_out.wait()

        @pl.loop(0, block_b // unroll)
        def _ctx_chunk(ci):
            off0 = base + ci * (unroll * W)
            rows = []
            for u in range(unroll):
                off = off0 + u * W
                ctx = [out_vmem[ids_ref[off + 1 + k], 0] for k in range(C)]
                rows.append(_tree_sum(ctx))
            for u in range(unroll):
                buf_ref[pl.ds(ci * unroll + u, 1), :] = rows[u][None, :]

        cp_in.wait()

        @pl.loop(0, block_b // unroll)
        def _tgt_chunk(ci):
            off0 = base + ci * (unroll * W)
            prods = []
            for u in range(unroll):
                off = off0 + u * W
                t = in_vmem[ids_ref[off], 0][None, :]
                prods.append(buf_ref[pl.ds(ci * unroll + u, 1), :] * t)
            for u in range(unroll):
                buf_ref[pl.ds(ci * unroll + u, 1), :] = prods[u]

        o_ref[...] = (jnp.sum(buf_ref[...], axis=-1) * inv_c)[None, :]

    return body


def _choose_block(B):
    if B >= 2048 and B % 2048 == 0:
        return B // 2
    if B >= 1024:
        return 512
    return max(_UNROLL, _round_up(B, _UNROLL))


def kernel(x, in_emb, out_emb):
    B, W = x.shape
    C = W - 1
    if C < 1:
        raise ValueError("Skipgram needs at least one context word (W >= 2).")
    V, H = in_emb.shape

    block_b = _choose_block(B)
    grid_b = -(-B // block_b)
    B_pad = grid_b * block_b

    x = x.astype(jnp.int32)
    if B_pad != B:
        x = jnp.pad(x, ((0, B_pad - B), (0, 0)))

    in3 = in_emb.reshape(V, 1, H)
    out3 = out_emb.reshape(V, 1, H)

    table_bytes = 2 * V * H * jnp.dtype(in_emb.dtype).itemsize
    vmem_need = table_bytes + block_b * H * 4 + block_b * 4

    out = pl.pallas_call(
        _make_kernel(block_b, W, H, _UNROLL),
        out_shape=jax.ShapeDtypeStruct((1, B_pad), jnp.float32),
        grid_spec=pltpu.PrefetchScalarGridSpec(
            num_scalar_prefetch=1,
            grid=(grid_b,),
            in_specs=[
                pl.BlockSpec(memory_space=pl.ANY),
                pl.BlockSpec(memory_space=pl.ANY),
            ],
            out_specs=pl.BlockSpec((1, block_b), lambda i, ids: (0, i)),
            scratch_shapes=[
                pltpu.VMEM((V, 1, H), jnp.float32),
                pltpu.VMEM((V, 1, H), jnp.float32),
                pltpu.VMEM((block_b, H), jnp.float32),
                pltpu.SemaphoreType.DMA,
                pltpu.SemaphoreType.DMA,
            ],
        ),
        compiler_params=pltpu.CompilerParams(
            dimension_semantics=("parallel",),
            vmem_limit_bytes=int(min(vmem_need + (16 << 20), 56 << 20)),
        ),
    )(x.reshape(-1), in3, out3)
    return out.reshape(B_pad)[:B]


# speedup vs baseline: 1.3220x; 1.0322x over previous
import jax
import jax.numpy as jnp
from jax.experimental import pallas as pl
from jax.experimental.pallas import tpu as pltpu

_UNROLL = 32


def _round_up(v, m):
    return ((v + m - 1) // m) * m


def _tree_sum(vals):
    vals = list(vals)
    while len(vals) > 1:
        nxt = [vals[i] + vals[i + 1] for i in range(0, len(vals) - 1, 2)]
        if len(vals) % 2:
            nxt.append(vals[-1])
        vals = nxt
    return vals[0]


def _make_kernel(block_b, W, H, unroll):
    C = W - 1
    inv_c = 1.0 / C

    def body(ids_ref, in_hbm, out_hbm, o_ref, in_vmem, out_vmem, buf_ref,
             sem_in, sem_out):
        blk = pl.program_id(0)
        base = blk * block_b * W

        cp_out = pltpu.make_async_copy(out_hbm, out_vmem, sem_out)
        cp_in = pltpu.make_async_copy(in_hbm, in_vmem, sem_in)
        cp_out.start()
        cp_in.start()
        cp_out.wait()

        @pl.loop(0, block_b // unroll)
        def _ctx_chunk(ci):
            off0 = base + ci * (unroll * W)
            rows = []
            for u in range(unroll):
                off = off0 + u * W
                ctx = [out_vmem[ids_ref[off + 1 + k], 0] for k in range(C)]
                rows.append(_tree_sum(ctx))
            for u in range(unroll):
                buf_ref[pl.ds(ci * unroll + u, 1), :] = rows[u][None, :]

        cp_in.wait()

        @pl.loop(0, block_b // unroll)
        def _tgt_chunk(ci):
            off0 = base + ci * (unroll * W)
            prods = []
            for u in range(unroll):
                off = off0 + u * W
                t = in_vmem[ids_ref[off], 0][None, :]
                prods.append(buf_ref[pl.ds(ci * unroll + u, 1), :] * t)
            for u in range(unroll):
                buf_ref[pl.ds(ci * unroll + u, 1), :] = prods[u]

        o_ref[...] = (jnp.sum(buf_ref[...], axis=-1) * inv_c)[None, :]

    return body


def _choose_block(B):
    if B >= 2048 and B % 2048 == 0:
        return B // 2
    if B >= 1024:
        return 512
    return max(_UNROLL, _round_up(B, _UNROLL))


def kernel(x, in_emb, out_emb):
    B, W = x.shape
    C = W - 1
    if C < 1:
        raise ValueError("Skipgram needs at least one context word (W >= 2).")
    V, H = in_emb.shape

    block_b = _choose_block(B)
    grid_b = -(-B // block_b)
    B_pad = grid_b * block_b

    x = x.astype(jnp.int32)
    if B_pad != B:
        x = jnp.pad(x, ((0, B_pad - B), (0, 0)))

    in3 = in_emb.reshape(V, 1, H)
    out3 = out_emb.reshape(V, 1, H)

    table_bytes = 2 * V * H * jnp.dtype(in_emb.dtype).itemsize
    vmem_need = table_bytes + block_b * H * 4 + block_b * 4

    out = pl.pallas_call(
        _make_kernel(block_b, W, H, _UNROLL),
        out_shape=jax.ShapeDtypeStruct((1, B_pad), jnp.float32),
        grid_spec=pltpu.PrefetchScalarGridSpec(
            num_scalar_prefetch=1,
            grid=(grid_b,),
            in_specs=[
                pl.BlockSpec(memory_space=pl.ANY),
                pl.BlockSpec(memory_space=pl.ANY),
            ],
            out_specs=pl.BlockSpec((1, block_b), lambda i, ids: (0, i)),
            scratch_shapes=[
                pltpu.VMEM((V, 1, H), jnp.float32),
                pltpu.VMEM((V, 1, H), jnp.float32),
                pltpu.VMEM((block_b, H), jnp.float32),
                pltpu.SemaphoreType.DMA,
                pltpu.SemaphoreType.DMA,
            ],
        ),
        compiler_params=pltpu.CompilerParams(
            dimension_semantics=("parallel",),
            vmem_limit_bytes=int(min(vmem_need + (16 << 20), 56 << 20)),
        ),
    )(x.reshape(-1), in3, out3)
    return out.reshape(B_pad)[:B]
